# baseline (speedup 1.0000x reference)
import functools

import numpy as np

import jax
import jax.numpy as jnp
from jax import lax
from jax.experimental import pallas as pl
from jax.experimental.pallas import tpu as pltpu
from jax.experimental.pallas import tpu_sc as plsc

NC = 2
NS = 16
LANES = 16
CHUNK = 128
NBUF = 2


def _chunking(e):
    for c in range(128, 15, -1):
        if e % (NC * NS * c) == 0 and (e // (NC * NS * c)) % NBUF == 0:
            return c, e // (NC * NS * c), 0
    g = 8 * NBUF // __import__("math").gcd(8, NBUF)
    tot = -(-e // (NC * NS * 128))
    tot = -(-tot // g) * g
    return 128, tot, NC * NS * tot * 128 - e


def _sc_scatter_partials(x, src_p, dst_p, n, d, cpr, chunk):
    rows_per_tile = -(-n // (NS * 8)) * 8
    n_pad = rows_per_tile * NS

    mesh = plsc.VectorSubcoreMesh(core_axis_name="c", subcore_axis_name="s")

    @functools.partial(
        pl.kernel,
        out_type=(jax.ShapeDtypeStruct((n_pad, d), jnp.float32),
                  jax.ShapeDtypeStruct((n_pad, d), jnp.float32)),
        mesh=mesh,
        scratch_types=[
            pltpu.VMEM((cpr, chunk), jnp.int32),
            pltpu.VMEM((cpr, chunk), jnp.int32),
            *[pltpu.VMEM((chunk, d), jnp.float32) for _ in range(NBUF)],
            pltpu.VMEM_SHARED((n_pad, d), jnp.float32),
            *[pltpu.SemaphoreType.DMA for _ in range(NBUF)],
        ],
    )
    def scatter_kernel(x_hbm, src_hbm, dst_hbm, out0_hbm, out1_hbm,
                       sidx, didx, *rest):
        rows = rest[:NBUF]
        acc = rest[NBUF]
        sems = rest[NBUF + 1:]
        cid = lax.axis_index("c")
        sid = lax.axis_index("s")

        w = cid * NS + sid
        pltpu.async_copy(src_hbm.at[w], sidx, sems[0])
        pltpu.async_copy(dst_hbm.at[w], didx, sems[1])

        zbuf = rows[0]
        def zero_body(t, _):
            zbuf[t // (d // LANES),
                 pl.ds((t % (d // LANES)) * LANES, LANES)] = (
                jnp.zeros((LANES,), jnp.float32))
            return 0
        zc = (chunk // 8) * 8
        lax.fori_loop(0, chunk * (d // LANES), zero_body, 0)
        r0 = sid * rows_per_tile
        full = rows_per_tile // zc
        for k in range(full):
            pltpu.sync_copy(zbuf.at[pl.ds(0, zc)],
                            acc.at[pl.ds(r0 + k * zc, zc)])
        rem = rows_per_tile - full * zc
        if rem:
            pltpu.sync_copy(zbuf.at[pl.ds(0, rem)],
                            acc.at[pl.ds(r0 + full * zc, rem)])
        pltpu.make_async_copy(src_hbm.at[w], sidx, sems[0]).wait()
        pltpu.make_async_copy(dst_hbm.at[w], didx, sems[1]).wait()
        plsc.subcore_barrier()

        for b in range(NBUF):
            pltpu.async_copy(x_hbm.at[sidx.at[b]], rows[b], sems[b])

        def pipe_body(jj, _):
            for b in range(NBUF):
                c = NBUF * jj + b
                pltpu.make_async_copy(
                    x_hbm.at[sidx.at[c]], rows[b], sems[b]).wait()
                pltpu.sync_copy(rows[b], acc.at[didx.at[c]], add=True)

                @pl.when(c + NBUF < cpr)
                def _prefetch():
                    pltpu.async_copy(x_hbm.at[sidx.at[c + NBUF]],
                                     rows[b], sems[b])
            return 0
        lax.fori_loop(0, cpr // NBUF, pipe_body, 0)
        plsc.subcore_barrier()

        @pl.when(cid == 0)
        def _out0():
            pltpu.sync_copy(acc.at[pl.ds(r0, rows_per_tile)],
                            out0_hbm.at[pl.ds(r0, rows_per_tile)])

        @pl.when(cid == 1)
        def _out1():
            pltpu.sync_copy(acc.at[pl.ds(r0, rows_per_tile)],
                            out1_hbm.at[pl.ds(r0, rows_per_tile)])

    return scatter_kernel(x, src_p, dst_p), n_pad


def _combine(x, p0, p1, n, d, scale):
    block = 2000

    def body(x_ref, a_ref, b_ref, o_ref):
        o_ref[...] = jnp.maximum(
            (x_ref[...] + a_ref[...] + b_ref[...]) * scale, 0.0)

    spec = pl.BlockSpec((block, d), lambda i: (i, 0))
    return pl.pallas_call(
        body,
        grid=(n // block,),
        in_specs=[spec, spec, spec],
        out_specs=spec,
        out_shape=jax.ShapeDtypeStruct((n, d), jnp.float32),
    )(x, p0, p1)


def kernel(x, edge_index, edge_weights, W_w, b_w, att):
    n, d = x.shape
    e = edge_index.shape[1]
    out_dim = att.shape[1]

    rows_per_tile = -(-n // (NS * 8)) * 8
    n_pad = rows_per_tile * NS
    chunk, cpr, pad = _chunking(e)

    if pad:
        spare = max(n_pad - n, 1)
        pad_src = jnp.asarray(np.arange(pad) % n, dtype=jnp.int32)
        pad_dst = jnp.asarray(n + (np.arange(pad) % spare),
                              dtype=jnp.int32)
        src_flat = jnp.concatenate([edge_index[0], pad_src])
        dst_flat = jnp.concatenate([edge_index[1], pad_dst])
    else:
        src_flat = edge_index[0]
        dst_flat = edge_index[1]
    src_p = src_flat.reshape(NC * NS, cpr, chunk)
    dst_p = dst_flat.reshape(NC * NS, cpr, chunk)

    (p0, p1), n_pad = _sc_scatter_partials(x, src_p, dst_p, n, d, cpr,
                                           chunk)
    return _combine(x, p0, p1, n, d, 1.0 / out_dim)


# speedup vs baseline: 1.0167x; 1.0167x over previous
import functools

import numpy as np

import jax
import jax.numpy as jnp
from jax import lax
from jax.experimental import pallas as pl
from jax.experimental.pallas import tpu as pltpu
from jax.experimental.pallas import tpu_sc as plsc

NC = 2
NS = 16
LANES = 16
CHUNK = 128
NBUF = 2


def _chunking(e):
    for c in (128,):
        if e % (NC * NS * c) == 0 and (e // (NC * NS * c)) % NBUF == 0:
            return c, e // (NC * NS * c), 0
    g = 8 * NBUF // __import__("math").gcd(8, NBUF)
    tot = -(-e // (NC * NS * 128))
    tot = -(-tot // g) * g
    return 128, tot, NC * NS * tot * 128 - e


def _sc_scatter_partials(x, src_p, dst_p, n, d, cpr, chunk):
    rows_per_tile = -(-n // (NS * 8)) * 8
    n_pad = rows_per_tile * NS

    mesh = plsc.VectorSubcoreMesh(core_axis_name="c", subcore_axis_name="s")

    @functools.partial(
        pl.kernel,
        out_type=(jax.ShapeDtypeStruct((n_pad, d), jnp.float32),
                  jax.ShapeDtypeStruct((n_pad, d), jnp.float32)),
        mesh=mesh,
        scratch_types=[
            pltpu.VMEM((cpr, chunk), jnp.int32),
            pltpu.VMEM((cpr, chunk), jnp.int32),
            *[pltpu.VMEM((chunk, d), jnp.float32) for _ in range(NBUF)],
            pltpu.VMEM_SHARED((n_pad, d), jnp.float32),
            *[pltpu.SemaphoreType.DMA for _ in range(NBUF)],
        ],
    )
    def scatter_kernel(x_hbm, src_hbm, dst_hbm, out0_hbm, out1_hbm,
                       sidx, didx, *rest):
        rows = rest[:NBUF]
        acc = rest[NBUF]
        sems = rest[NBUF + 1:]
        cid = lax.axis_index("c")
        sid = lax.axis_index("s")

        w = cid * NS + sid
        pltpu.async_copy(src_hbm.at[w], sidx, sems[0])
        pltpu.async_copy(dst_hbm.at[w], didx, sems[1])

        zbuf = rows[0]
        def zero_body(t, _):
            zbuf[t // (d // LANES),
                 pl.ds((t % (d // LANES)) * LANES, LANES)] = (
                jnp.zeros((LANES,), jnp.float32))
            return 0
        zc = (chunk // 8) * 8
        lax.fori_loop(0, chunk * (d // LANES), zero_body, 0)
        r0 = sid * rows_per_tile
        full = rows_per_tile // zc
        for k in range(full):
            pltpu.sync_copy(zbuf.at[pl.ds(0, zc)],
                            acc.at[pl.ds(r0 + k * zc, zc)])
        rem = rows_per_tile - full * zc
        if rem:
            pltpu.sync_copy(zbuf.at[pl.ds(0, rem)],
                            acc.at[pl.ds(r0 + full * zc, rem)])
        pltpu.make_async_copy(src_hbm.at[w], sidx, sems[0]).wait()
        pltpu.make_async_copy(dst_hbm.at[w], didx, sems[1]).wait()
        plsc.subcore_barrier()

        for b in range(NBUF):
            pltpu.async_copy(x_hbm.at[sidx.at[b]], rows[b], sems[b])

        def pipe_body(jj, _):
            for b in range(NBUF):
                c = NBUF * jj + b
                pltpu.make_async_copy(
                    x_hbm.at[sidx.at[c]], rows[b], sems[b]).wait()
                pltpu.sync_copy(rows[b], acc.at[didx.at[c]], add=True)

                @pl.when(c + NBUF < cpr)
                def _prefetch():
                    pltpu.async_copy(x_hbm.at[sidx.at[c + NBUF]],
                                     rows[b], sems[b])
            return 0
        lax.fori_loop(0, cpr // NBUF, pipe_body, 0)
        plsc.subcore_barrier()

        @pl.when(cid == 0)
        def _out0():
            pltpu.sync_copy(acc.at[pl.ds(r0, rows_per_tile)],
                            out0_hbm.at[pl.ds(r0, rows_per_tile)])

        @pl.when(cid == 1)
        def _out1():
            pltpu.sync_copy(acc.at[pl.ds(r0, rows_per_tile)],
                            out1_hbm.at[pl.ds(r0, rows_per_tile)])

    return scatter_kernel(x, src_p, dst_p), n_pad


def _combine(x, p0, p1, n, d, scale):
    block = 2000

    def body(x_ref, a_ref, b_ref, o_ref):
        o_ref[...] = jnp.maximum(
            (x_ref[...] + a_ref[...] + b_ref[...]) * scale, 0.0)

    spec = pl.BlockSpec((block, d), lambda i: (i, 0))
    return pl.pallas_call(
        body,
        grid=(n // block,),
        in_specs=[spec, spec, spec],
        out_specs=spec,
        out_shape=jax.ShapeDtypeStruct((n, d), jnp.float32),
    )(x, p0, p1)


def kernel(x, edge_index, edge_weights, W_w, b_w, att):
    n, d = x.shape
    e = edge_index.shape[1]
    out_dim = att.shape[1]

    rows_per_tile = -(-n // (NS * 8)) * 8
    n_pad = rows_per_tile * NS
    chunk, cpr, pad = _chunking(e)

    if pad:
        spare = max(n_pad - n, 1)
        pad_src = jnp.asarray(np.arange(pad) % n, dtype=jnp.int32)
        pad_dst = jnp.asarray(n + (np.arange(pad) % spare),
                              dtype=jnp.int32)
        src_flat = jnp.concatenate([edge_index[0], pad_src])
        dst_flat = jnp.concatenate([edge_index[1], pad_dst])
    else:
        src_flat = edge_index[0]
        dst_flat = edge_index[1]
    src_p = src_flat.reshape(NC * NS, cpr, chunk)
    dst_p = dst_flat.reshape(NC * NS, cpr, chunk)

    (p0, p1), n_pad = _sc_scatter_partials(x, src_p, dst_p, n, d, cpr,
                                           chunk)
    return _combine(x, p0, p1, n, d, 1.0 / out_dim)
